# baseline (speedup 1.0000x reference)
import functools

import jax
import jax.numpy as jnp
from jax import lax
from jax.experimental import pallas as pl
from jax.experimental.pallas import tpu as pltpu
from jax.experimental.pallas import tpu_sc as plsc

RES = 2048
M = RES * RES
N = 1048576
L = 16
NC, NS = 2, 16
NW = NC * NS
PER_W = N // NW
C = 2048
K = C // 128
NCHUNK = PER_W // C
NPAIR = NCHUNK // 2
NTILE = (RES // 8) * (RES // 128)
TPW = NTILE // NW
TSZ = 8 * 128

_mesh = plsc.VectorSubcoreMesh(core_axis_name="c", subcore_axis_name="s")


@functools.partial(
    pl.kernel,
    mesh=_mesh,
    compiler_params=pltpu.CompilerParams(
        needs_layout_passes=False, use_tc_tiling_on_sc=False),
    out_type=jax.ShapeDtypeStruct((8 * M,), jnp.float32),
    scratch_types=[
        pltpu.VMEM((3 * (TSZ + 128),), jnp.float32),
        pltpu.VMEM((3 * (TSZ + 128),), jnp.float32),
        pltpu.VMEM((8 * TSZ,), jnp.float32),
        pltpu.VMEM((8 * TSZ,), jnp.float32),
        pltpu.SemaphoreType.DMA,
        pltpu.SemaphoreType.DMA,
        pltpu.SemaphoreType.DMA,
        pltpu.SemaphoreType.DMA,
    ],
)
def _build_kernel(table_hbm, t8_hbm, cb0, cb1, ob0, ob1,
                  sem_i0, sem_i1, sem_o0, sem_o1):
    wid = lax.axis_index("s") * NC + lax.axis_index("c")
    t0 = wid * TPW
    lane = lax.iota(jnp.int32, L)
    sem_i = {id(cb0): sem_i0, id(cb1): sem_i1}
    sem_o = {id(ob0): sem_o0, id(ob1): sem_o1}

    CSZ = TSZ + 128

    def fetch(ti, cb):
        off = ti * TSZ
        ei = jnp.where(ti + 16 < NTILE, ti + 16, ti) * TSZ
        for c in range(3):
            pltpu.async_copy(table_hbm.at[pl.ds(c * M + off, TSZ)],
                             cb.at[pl.ds(c * CSZ, TSZ)], sem_i[id(cb)])
            pltpu.async_copy(table_hbm.at[pl.ds(c * M + ei, 128)],
                             cb.at[pl.ds(c * CSZ + TSZ, 128)], sem_i[id(cb)])

    def drain_fetch(cb):
        for c in range(3):
            pltpu.make_async_copy(table_hbm.at[pl.ds(0, TSZ)],
                                  cb.at[pl.ds(c * CSZ, TSZ)],
                                  sem_i[id(cb)]).wait()
            pltpu.make_async_copy(table_hbm.at[pl.ds(0, 128)],
                                  cb.at[pl.ds(c * CSZ + TSZ, 128)],
                                  sem_i[id(cb)]).wait()

    def shuffle(cb, ob):
        @plsc.parallel_loop(0, TSZ // L, unroll=2)
        def body(v):
            s = v * L
            idx8 = (s + lane) * 8
            for c in range(3):
                lo = cb[pl.ds(c * CSZ + s, L)]
                hi = cb[pl.ds(c * CSZ + s + 128, L)]
                plsc.store_scatter(ob, [idx8 + c], lo)
                plsc.store_scatter(ob, [idx8 + (4 + c)], hi)

    def emit(ti, ob):
        ty = ti >> 4
        tx = ti & 15
        for ry in range(8):
            dst = (ty * 8 + ry) * (RES * 8) + tx * 1024
            pltpu.async_copy(ob.at[pl.ds(ry * 1024, 1024)],
                             t8_hbm.at[pl.ds(dst, 1024)], sem_o[id(ob)])

    def drain_emit(ob):
        for ry in range(8):
            pltpu.make_async_copy(ob.at[pl.ds(ry * 1024, 1024)],
                                  t8_hbm.at[pl.ds(0, 1024)],
                                  sem_o[id(ob)]).wait()

    fetch(t0, cb0)
    fetch(t0 + 1, cb1)
    drain_fetch(cb0)
    shuffle(cb0, ob0)
    fetch(t0 + 2, cb0)
    emit(t0, ob0)
    drain_fetch(cb1)
    shuffle(cb1, ob1)
    fetch(t0 + 3, cb1)
    emit(t0 + 1, ob1)

    def pair_body(j, carry):
        ti = t0 + 2 * j
        drain_fetch(cb0)
        drain_emit(ob0)
        shuffle(cb0, ob0)

        @pl.when(2 * j + 2 < TPW)
        def _():
            fetch(ti + 2, cb0)

        emit(ti, ob0)
        drain_fetch(cb1)
        drain_emit(ob1)
        shuffle(cb1, ob1)

        @pl.when(2 * j + 3 < TPW)
        def _():
            fetch(ti + 3, cb1)

        emit(ti + 1, ob1)
        return carry

    lax.fori_loop(1, TPW // 2, pair_body, 0)
    drain_emit(ob0)
    drain_emit(ob1)


@functools.partial(
    pl.kernel,
    mesh=_mesh,
    compiler_params=pltpu.CompilerParams(
        needs_layout_passes=False, use_tc_tiling_on_sc=False),
    out_type=jax.ShapeDtypeStruct((3 * N,), jnp.float32),
    scratch_types=[
        pltpu.VMEM((2, C), jnp.float32),
        pltpu.VMEM((2, C), jnp.float32),
        pltpu.VMEM((2, C), jnp.float32),
        pltpu.VMEM((2, C), jnp.float32),
        pltpu.VMEM((2, 2, K, 128), jnp.int32),
        pltpu.VMEM((C, 8), jnp.float32),
        pltpu.VMEM((C, 8), jnp.float32),
        pltpu.VMEM((C, 8), jnp.float32),
        pltpu.VMEM((C, 8), jnp.float32),
        pltpu.VMEM((3 * C,), jnp.float32),
        pltpu.VMEM((3 * C,), jnp.float32),
        pltpu.SemaphoreType.DMA,
        pltpu.SemaphoreType.DMA,
        pltpu.SemaphoreType.DMA,
        pltpu.SemaphoreType.DMA,
        pltpu.SemaphoreType.DMA,
    ],
)
def _grid_kernel(xs_hbm, ys_hbm, t8_hbm, out_hbm,
                 xs_v, ys_v, wx_v, wy_v, idx_v,
                 g0a, g1a, g0b, g1b,
                 outa, outb, sem_a, sem_b, sem_i0, sem_i1, sem_o):
    wid = lax.axis_index("s") * NC + lax.axis_index("c")
    base = wid * PER_W
    lane = lax.iota(jnp.int32, L)
    gbufs = ((g0a, g1a), (g0b, g1b))
    outs = (outa, outb)
    sems = (sem_a, sem_b)
    chans = tuple(jnp.full((L,), c, jnp.int32) for c in range(3))
    chans_hi = tuple(jnp.full((L,), 4 + c, jnp.int32) for c in range(3))

    sem_i = (sem_i0, sem_i1)

    def prefetch_xy(ci, b):
        off = base + ci * C
        pltpu.async_copy(xs_hbm.at[pl.ds(off, C)], xs_v.at[b], sem_i[b])
        pltpu.async_copy(ys_hbm.at[pl.ds(off, C)], ys_v.at[b], sem_i[b])

    def caf(ci, b):
        pltpu.make_async_copy(
            xs_hbm.at[pl.ds(0, C)], xs_v.at[b], sem_i[b]).wait()
        pltpu.make_async_copy(
            ys_hbm.at[pl.ds(0, C)], ys_v.at[b], sem_i[b]).wait()

        @plsc.parallel_loop(0, C // L, unroll=2)
        def idx_body(v):
            s = v * L
            x = xs_v[b, pl.ds(s, L)] * (RES - 1.0)
            xi = jnp.minimum(jnp.maximum(x.astype(jnp.int32), 0), RES - 2)
            tx = x - xi.astype(jnp.float32)
            y = ys_v[b, pl.ds(s, L)] * (RES - 1.0)
            yi = jnp.minimum(jnp.maximum(y.astype(jnp.int32), 0), RES - 2)
            ty = y - yi.astype(jnp.float32)
            wx_v[b, pl.ds(s, L)] = tx
            wy_v[b, pl.ds(s, L)] = ty
            r00 = yi * RES + xi
            k = v // 8
            col = (v % 8) * L
            idx_v[b, 0, k, pl.ds(col, L)] = r00
            idx_v[b, 1, k, pl.ds(col, L)] = r00 + 1

        @pl.when(ci + 2 < NCHUNK)
        def _():
            prefetch_xy(ci + 2, b)

        for k in range(K):
            for t in range(2):
                pltpu.async_copy(
                    t8_hbm.at[idx_v.at[b, t, k]],
                    gbufs[b][t].at[pl.ds(k * 128, 128), :],
                    sems[b])

    def drain_gathers(b):
        for t in range(2):
            pltpu.make_async_copy(
                t8_hbm.at[pl.ds(0, C), :], gbufs[b][t], sems[b]).wait()

    def combine(ci, b):
        g0, g1 = gbufs[b]
        out_v = outs[b]

        @plsc.parallel_loop(0, C // L, unroll=4)
        def comb_body(j):
            s = j * L
            idxp = s + lane
            tx = wx_v[b, pl.ds(s, L)]
            ty = wy_v[b, pl.ds(s, L)]
            ux = 1.0 - tx
            uy = 1.0 - ty
            w00 = ux * uy
            w01 = tx * uy
            w10 = ux * ty
            w11 = tx * ty
            for cc in range(3):
                a = plsc.load_gather(g0, [idxp, chans[cc]])
                b2 = plsc.load_gather(g1, [idxp, chans[cc]])
                d2 = plsc.load_gather(g0, [idxp, chans_hi[cc]])
                e2 = plsc.load_gather(g1, [idxp, chans_hi[cc]])
                out_v[pl.ds(cc * C + s, L)] = (
                    a * w00 + b2 * w01 + d2 * w10 + e2 * w11)

        off = base + ci * C
        return [pltpu.async_copy(out_v.at[pl.ds(cc * C, C)],
                                 out_hbm.at[pl.ds(cc * N + off, C)], sem_o)
                for cc in range(3)]

    prefetch_xy(0, 0)
    prefetch_xy(1, 1)
    caf(0, 0)

    def pair_body(j, carry):
        ci = 2 * j
        caf(ci + 1, 1)
        drain_gathers(0)
        cps_a = combine(ci, 0)

        @pl.when(ci + 2 < NCHUNK)
        def _():
            caf(ci + 2, 0)

        drain_gathers(1)
        cps_b = combine(ci + 1, 1)
        for cp in cps_a + cps_b:
            cp.wait()
        return carry

    lax.fori_loop(0, NPAIR, pair_body, 0)


def kernel(input, feature_params):
    xs = input[:, 0]
    ys = input[:, 1]
    table = (feature_params.transpose(2, 0, 1)
             .reshape(3, RES // 8, 8, RES // 128, 128)
             .transpose(0, 1, 3, 2, 4)
             .reshape(3 * M))
    t8 = _build_kernel(table).reshape(M, 8)
    out_flat = _grid_kernel(xs, ys, t8)
    return out_flat.reshape(3, N).transpose(1, 0)


# speedup vs baseline: 1.0138x; 1.0138x over previous
import functools

import jax
import jax.numpy as jnp
from jax import lax
from jax.experimental import pallas as pl
from jax.experimental.pallas import tpu as pltpu
from jax.experimental.pallas import tpu_sc as plsc

RES = 2048
M = RES * RES
N = 1048576
L = 16
NC, NS = 2, 16
NW = NC * NS
PER_W = N // NW
C = 1024
K = C // 128
NCHUNK = PER_W // C
NPAIR = NCHUNK // 2
NTILE = (RES // 8) * (RES // 128)
TPW = NTILE // NW
TSZ = 8 * 128

_mesh = plsc.VectorSubcoreMesh(core_axis_name="c", subcore_axis_name="s")


@functools.partial(
    pl.kernel,
    mesh=_mesh,
    compiler_params=pltpu.CompilerParams(
        needs_layout_passes=False, use_tc_tiling_on_sc=False),
    out_type=jax.ShapeDtypeStruct((8 * M,), jnp.float32),
    scratch_types=[
        pltpu.VMEM((3 * (TSZ + 128),), jnp.float32),
        pltpu.VMEM((3 * (TSZ + 128),), jnp.float32),
        pltpu.VMEM((8 * TSZ,), jnp.float32),
        pltpu.VMEM((8 * TSZ,), jnp.float32),
        pltpu.SemaphoreType.DMA,
        pltpu.SemaphoreType.DMA,
        pltpu.SemaphoreType.DMA,
        pltpu.SemaphoreType.DMA,
    ],
)
def _build_kernel(table_hbm, t8_hbm, cb0, cb1, ob0, ob1,
                  sem_i0, sem_i1, sem_o0, sem_o1):
    wid = lax.axis_index("s") * NC + lax.axis_index("c")
    t0 = wid * TPW
    lane = lax.iota(jnp.int32, L)
    sem_i = {id(cb0): sem_i0, id(cb1): sem_i1}
    sem_o = {id(ob0): sem_o0, id(ob1): sem_o1}

    CSZ = TSZ + 128

    def fetch(ti, cb):
        off = ti * TSZ
        ei = jnp.where(ti + 16 < NTILE, ti + 16, ti) * TSZ
        for c in range(3):
            pltpu.async_copy(table_hbm.at[pl.ds(c * M + off, TSZ)],
                             cb.at[pl.ds(c * CSZ, TSZ)], sem_i[id(cb)])
            pltpu.async_copy(table_hbm.at[pl.ds(c * M + ei, 128)],
                             cb.at[pl.ds(c * CSZ + TSZ, 128)], sem_i[id(cb)])

    def drain_fetch(cb):
        for c in range(3):
            pltpu.make_async_copy(table_hbm.at[pl.ds(0, TSZ)],
                                  cb.at[pl.ds(c * CSZ, TSZ)],
                                  sem_i[id(cb)]).wait()
            pltpu.make_async_copy(table_hbm.at[pl.ds(0, 128)],
                                  cb.at[pl.ds(c * CSZ + TSZ, 128)],
                                  sem_i[id(cb)]).wait()

    def shuffle(cb, ob):
        @plsc.parallel_loop(0, TSZ // L, unroll=2)
        def body(v):
            s = v * L
            idx8 = (s + lane) * 8
            for c in range(3):
                lo = cb[pl.ds(c * CSZ + s, L)]
                hi = cb[pl.ds(c * CSZ + s + 128, L)]
                plsc.store_scatter(ob, [idx8 + c], lo)
                plsc.store_scatter(ob, [idx8 + (4 + c)], hi)

    def emit(ti, ob):
        ty = ti >> 4
        tx = ti & 15
        for ry in range(8):
            dst = (ty * 8 + ry) * (RES * 8) + tx * 1024
            pltpu.async_copy(ob.at[pl.ds(ry * 1024, 1024)],
                             t8_hbm.at[pl.ds(dst, 1024)], sem_o[id(ob)])

    def drain_emit(ob):
        for ry in range(8):
            pltpu.make_async_copy(ob.at[pl.ds(ry * 1024, 1024)],
                                  t8_hbm.at[pl.ds(0, 1024)],
                                  sem_o[id(ob)]).wait()

    fetch(t0, cb0)
    fetch(t0 + 1, cb1)
    drain_fetch(cb0)
    shuffle(cb0, ob0)
    fetch(t0 + 2, cb0)
    emit(t0, ob0)
    drain_fetch(cb1)
    shuffle(cb1, ob1)
    fetch(t0 + 3, cb1)
    emit(t0 + 1, ob1)

    def pair_body(j, carry):
        ti = t0 + 2 * j
        drain_fetch(cb0)
        drain_emit(ob0)
        shuffle(cb0, ob0)

        @pl.when(2 * j + 2 < TPW)
        def _():
            fetch(ti + 2, cb0)

        emit(ti, ob0)
        drain_fetch(cb1)
        drain_emit(ob1)
        shuffle(cb1, ob1)

        @pl.when(2 * j + 3 < TPW)
        def _():
            fetch(ti + 3, cb1)

        emit(ti + 1, ob1)
        return carry

    lax.fori_loop(1, TPW // 2, pair_body, 0)
    drain_emit(ob0)
    drain_emit(ob1)


@functools.partial(
    pl.kernel,
    mesh=_mesh,
    compiler_params=pltpu.CompilerParams(
        needs_layout_passes=False, use_tc_tiling_on_sc=False),
    out_type=jax.ShapeDtypeStruct((3 * N,), jnp.float32),
    scratch_types=[
        pltpu.VMEM((2, C), jnp.float32),
        pltpu.VMEM((2, C), jnp.float32),
        pltpu.VMEM((2, C), jnp.float32),
        pltpu.VMEM((2, C), jnp.float32),
        pltpu.VMEM((2, 2, K, 128), jnp.int32),
        pltpu.VMEM((C, 8), jnp.float32),
        pltpu.VMEM((C, 8), jnp.float32),
        pltpu.VMEM((C, 8), jnp.float32),
        pltpu.VMEM((C, 8), jnp.float32),
        pltpu.VMEM((3 * C,), jnp.float32),
        pltpu.VMEM((3 * C,), jnp.float32),
        pltpu.SemaphoreType.DMA,
        pltpu.SemaphoreType.DMA,
        pltpu.SemaphoreType.DMA,
        pltpu.SemaphoreType.DMA,
        pltpu.SemaphoreType.DMA,
    ],
)
def _grid_kernel(xs_hbm, ys_hbm, t8_hbm, out_hbm,
                 xs_v, ys_v, wx_v, wy_v, idx_v,
                 g0a, g1a, g0b, g1b,
                 outa, outb, sem_a, sem_b, sem_i0, sem_i1, sem_o):
    wid = lax.axis_index("s") * NC + lax.axis_index("c")
    base = wid * PER_W
    lane = lax.iota(jnp.int32, L)
    gbufs = ((g0a, g1a), (g0b, g1b))
    outs = (outa, outb)
    sems = (sem_a, sem_b)
    chans = tuple(jnp.full((L,), c, jnp.int32) for c in range(3))
    chans_hi = tuple(jnp.full((L,), 4 + c, jnp.int32) for c in range(3))

    sem_i = (sem_i0, sem_i1)

    def prefetch_xy(ci, b):
        off = base + ci * C
        pltpu.async_copy(xs_hbm.at[pl.ds(off, C)], xs_v.at[b], sem_i[b])
        pltpu.async_copy(ys_hbm.at[pl.ds(off, C)], ys_v.at[b], sem_i[b])

    def caf(ci, b):
        pltpu.make_async_copy(
            xs_hbm.at[pl.ds(0, C)], xs_v.at[b], sem_i[b]).wait()
        pltpu.make_async_copy(
            ys_hbm.at[pl.ds(0, C)], ys_v.at[b], sem_i[b]).wait()

        @plsc.parallel_loop(0, C // L, unroll=2)
        def idx_body(v):
            s = v * L
            x = xs_v[b, pl.ds(s, L)] * (RES - 1.0)
            xi = jnp.minimum(jnp.maximum(x.astype(jnp.int32), 0), RES - 2)
            tx = x - xi.astype(jnp.float32)
            y = ys_v[b, pl.ds(s, L)] * (RES - 1.0)
            yi = jnp.minimum(jnp.maximum(y.astype(jnp.int32), 0), RES - 2)
            ty = y - yi.astype(jnp.float32)
            wx_v[b, pl.ds(s, L)] = tx
            wy_v[b, pl.ds(s, L)] = ty
            r00 = yi * RES + xi
            k = v // 8
            col = (v % 8) * L
            idx_v[b, 0, k, pl.ds(col, L)] = r00
            idx_v[b, 1, k, pl.ds(col, L)] = r00 + 1

        @pl.when(ci + 2 < NCHUNK)
        def _():
            prefetch_xy(ci + 2, b)

        for k in range(K):
            for t in range(2):
                pltpu.async_copy(
                    t8_hbm.at[idx_v.at[b, t, k]],
                    gbufs[b][t].at[pl.ds(k * 128, 128), :],
                    sems[b])

    def drain_gathers(b):
        for t in range(2):
            pltpu.make_async_copy(
                t8_hbm.at[pl.ds(0, C), :], gbufs[b][t], sems[b]).wait()

    def combine(ci, b):
        g0, g1 = gbufs[b]
        out_v = outs[b]

        @plsc.parallel_loop(0, C // L, unroll=2)
        def comb_body(j):
            s = j * L
            idxp = s + lane
            tx = wx_v[b, pl.ds(s, L)]
            ty = wy_v[b, pl.ds(s, L)]
            ux = 1.0 - tx
            uy = 1.0 - ty
            w00 = ux * uy
            w01 = tx * uy
            w10 = ux * ty
            w11 = tx * ty
            for cc in range(3):
                a = plsc.load_gather(g0, [idxp, chans[cc]])
                b2 = plsc.load_gather(g1, [idxp, chans[cc]])
                d2 = plsc.load_gather(g0, [idxp, chans_hi[cc]])
                e2 = plsc.load_gather(g1, [idxp, chans_hi[cc]])
                out_v[pl.ds(cc * C + s, L)] = (
                    a * w00 + b2 * w01 + d2 * w10 + e2 * w11)

        off = base + ci * C
        return [pltpu.async_copy(out_v.at[pl.ds(cc * C, C)],
                                 out_hbm.at[pl.ds(cc * N + off, C)], sem_o)
                for cc in range(3)]

    prefetch_xy(0, 0)
    prefetch_xy(1, 1)
    caf(0, 0)

    def pair_body(j, carry):
        ci = 2 * j
        caf(ci + 1, 1)
        drain_gathers(0)
        cps_a = combine(ci, 0)

        @pl.when(ci + 2 < NCHUNK)
        def _():
            caf(ci + 2, 0)

        drain_gathers(1)
        cps_b = combine(ci + 1, 1)
        for cp in cps_a + cps_b:
            cp.wait()
        return carry

    lax.fori_loop(0, NPAIR, pair_body, 0)


def kernel(input, feature_params):
    xs = input[:, 0]
    ys = input[:, 1]
    table = (feature_params.transpose(2, 0, 1)
             .reshape(3, RES // 8, 8, RES // 128, 128)
             .transpose(0, 1, 3, 2, 4)
             .reshape(3 * M))
    t8 = _build_kernel(table).reshape(M, 8)
    out_flat = _grid_kernel(xs, ys, t8)
    return out_flat.reshape(3, N).transpose(1, 0)
